# baseline (speedup 1.0000x reference)
import functools

import numpy as np
import jax
import jax.numpy as jnp
from jax import lax
from jax.experimental import pallas as pl
from jax.experimental.pallas import tpu as pltpu
from jax.experimental.pallas import tpu_sc as plsc

_SEQ_LENGTH = 8192
_MASK_LENGTH = 2048
_D = 1024
_B = 4

_np_rng = np.random.RandomState(0)
_unmask_draw = _np_rng.randint(low=0, high=_SEQ_LENGTH, size=_MASK_LENGTH)
_UNMASK_BOOL = np.zeros(_SEQ_LENGTH, dtype=bool)
_UNMASK_BOOL[_unmask_draw] = True
_KEEP = np.where(_UNMASK_BOOL)[0].astype(np.int32)
_K = int(_KEEP.shape[0])

_ROWS = (np.arange(_B, dtype=np.int32)[:, None] * _SEQ_LENGTH
         + _KEEP[None, :]).reshape(-1)

_info = plsc.get_sparse_core_info()
_NC = _info.num_cores
_NS = _info.num_subcores
_NW = _NC * _NS

_CHUNK = 40
_N_CHUNKS = 6
_ROWS_PER_W = _CHUNK * _N_CHUNKS
_B_PAD = _ROWS_PER_W * _NW

_ROWS_PADDED = np.zeros(_B_PAD, dtype=np.int32)
_ROWS_PADDED[:_ROWS.shape[0]] = _ROWS
_IDX_TABLE = _ROWS_PADDED.reshape(_NW, _N_CHUNKS, _CHUNK)

_mesh = plsc.VectorSubcoreMesh(core_axis_name="c", subcore_axis_name="s")


@functools.partial(
    pl.kernel,
    mesh=_mesh,
    out_type=jax.ShapeDtypeStruct((_B_PAD, _D), jnp.float32),
    scratch_types=[
        pltpu.VMEM((_N_CHUNKS, _CHUNK), jnp.int32),
        pltpu.VMEM((_CHUNK, _D), jnp.float32),
        pltpu.VMEM((_CHUNK, _D), jnp.float32),
        pltpu.SemaphoreType.DMA,
        pltpu.SemaphoreType.DMA,
    ],
)
def _gather_rows(x_hbm, idx_hbm, out_hbm, idx_v, buf0, buf1, sem0, sem1):
    wid = lax.axis_index("s") * _NC + lax.axis_index("c")
    base = wid * _ROWS_PER_W
    pltpu.sync_copy(idx_hbm.at[wid], idx_v)
    bufs = (buf0, buf1)
    sems = (sem0, sem1)
    copies = [None, None]
    copies[0] = pltpu.async_copy(x_hbm.at[idx_v.at[0]], buf0, sem0)
    for c in range(_N_CHUNKS):
        nxt = (c + 1) % 2
        cur = c % 2
        if c + 1 < _N_CHUNKS:
            copies[nxt] = pltpu.async_copy(
                x_hbm.at[idx_v.at[c + 1]], bufs[nxt], sems[nxt])
        copies[cur].wait()
        pltpu.sync_copy(bufs[cur], out_hbm.at[pl.ds(base + c * _CHUNK, _CHUNK)])


def kernel(x):
    x_flat = x.reshape(_B * _SEQ_LENGTH, _D)
    idx = jnp.asarray(_IDX_TABLE)
    out_flat = _gather_rows(x_flat, idx)
    out = out_flat[: _B * _K].reshape(_B, _K, _D)
    return (out, jnp.asarray(_UNMASK_BOOL))
